# Initial kernel scaffold; baseline (speedup 1.0000x reference)
import jax
import jax.numpy as jnp
from jax.experimental import pallas as pl


def kernel(node_emb, w0, root0, b0, w1, root1, b1, edge_index, edge_type):
    raise NotImplementedError("write your pallas kernel here")


# speedup vs baseline: 5.4990x; 5.4990x over previous
import functools

import jax
import jax.numpy as jnp
from jax import lax
from jax.experimental import pallas as pl
from jax.experimental.pallas import tpu as pltpu
from jax.experimental.pallas import tpu_sc as plsc

N = 10000
R = 8
H = 160
NB = 5
BS = H // NB
E = 320000
NR = N * R

NC = 2
NS = 16
NW = NC * NS
LANES = 16

CH = 128
NCH = E // CH
ROWS_PER_TILE = N // NS
HV = H // LANES

_MESH = plsc.VectorSubcoreMesh(core_axis_name="c", subcore_axis_name="s")


def _worker_id():
    return lax.axis_index("c") * NS + lax.axis_index("s")


def _chunk_loop(body):
    wid = _worker_id()
    n_my = (NCH - wid + NW - 1) // NW

    def outer(i, carry):
        body(wid + i * NW)
        return carry

    lax.fori_loop(0, n_my, outer, 0)


@functools.partial(
    pl.kernel,
    out_type=(
        jax.ShapeDtypeStruct((E,), jnp.int32),
        jax.ShapeDtypeStruct((E,), jnp.int32),
        jax.ShapeDtypeStruct((NC * NR,), jnp.float32),
    ),
    mesh=_MESH,
    scratch_types=[
        pltpu.VMEM((CH,), jnp.int32),
        pltpu.VMEM((CH,), jnp.int32),
        pltpu.VMEM((CH,), jnp.int32),
        pltpu.VMEM((CH,), jnp.int32),
        pltpu.VMEM((CH,), jnp.int32),
        pltpu.VMEM((CH,), jnp.float32),
        pltpu.VMEM((CH,), jnp.float32),
        pltpu.VMEM_SHARED((NR,), jnp.float32),
    ],
    compiler_params=pltpu.CompilerParams(use_tc_tiling_on_sc=False),
)
def _p1(src_hbm, dst_hbm, typ_hbm, gidx_hbm, comb_hbm, cnt_hbm,
        sbuf, dbuf, tbuf, gbuf, cbuf, ones, zbuf, cnt_sh):
    cid = lax.axis_index("c")
    sid = lax.axis_index("s")

    for v in range(CH // LANES):
        ones[pl.ds(v * LANES, LANES)] = jnp.ones((LANES,), jnp.float32)
        zbuf[pl.ds(v * LANES, LANES)] = jnp.zeros((LANES,), jnp.float32)
    zrow = sid * (NR // NS)

    def zero_strip(i, carry):
        pltpu.sync_copy(zbuf.at[pl.ds(0, CH)],
                        cnt_sh.at[pl.ds(zrow + i * CH, CH)])
        return carry

    lax.fori_loop(0, (NR // NS) // CH, zero_strip, 0)
    pltpu.sync_copy(zbuf.at[pl.ds(0, 8)],
                    cnt_sh.at[pl.ds(zrow + ((NR // NS) // CH) * CH, 8)])
    plsc.subcore_barrier()

    def chunk(c):
        base = c * CH
        pltpu.sync_copy(src_hbm.at[pl.ds(base, CH)], sbuf)
        pltpu.sync_copy(dst_hbm.at[pl.ds(base, CH)], dbuf)
        pltpu.sync_copy(typ_hbm.at[pl.ds(base, CH)], tbuf)

        def vec(v, carry):
            sl = pl.ds(v * LANES, LANES)
            s = sbuf[sl]
            d = dbuf[sl]
            t = tbuf[sl]
            gbuf[sl] = t * N + s
            cbuf[sl] = d * R + t
            return carry

        lax.fori_loop(0, CH // LANES, vec, 0)
        pltpu.sync_copy(gbuf, gidx_hbm.at[pl.ds(base, CH)])
        pltpu.sync_copy(cbuf, comb_hbm.at[pl.ds(base, CH)])
        pltpu.sync_copy(ones, cnt_sh.at[cbuf], add=True)

    _chunk_loop(chunk)
    plsc.subcore_barrier()

    def dump(i, carry):
        pltpu.sync_copy(cnt_sh.at[pl.ds(zrow + i * CH, CH)], zbuf)
        pltpu.sync_copy(zbuf, cnt_hbm.at[pl.ds(cid * NR + zrow + i * CH, CH)])
        return carry

    lax.fori_loop(0, (NR // NS) // CH, dump, 0)
    tail = ((NR // NS) // CH) * CH
    pltpu.sync_copy(cnt_sh.at[pl.ds(zrow + tail, 8)], zbuf.at[pl.ds(0, 8)])
    pltpu.sync_copy(zbuf.at[pl.ds(0, 8)],
                    cnt_hbm.at[pl.ds(cid * NR + zrow + tail, 8)])


def _p2_body(cnt_ref, inv_ref):
    c = cnt_ref[0] + cnt_ref[1]
    inv_ref[...] = 1.0 / jnp.maximum(c, 1.0)


def _p2(cnt):
    cnt3 = cnt.reshape(NC, NR // 128, 128)
    inv = pl.pallas_call(
        _p2_body,
        out_shape=jax.ShapeDtypeStruct((NR // 128, 128), jnp.float32),
    )(cnt3)
    return inv.reshape(NR)


@functools.partial(
    pl.kernel,
    out_type=jax.ShapeDtypeStruct((E,), jnp.float32),
    mesh=_MESH,
    scratch_types=[
        pltpu.VMEM((CH,), jnp.int32),
        pltpu.VMEM((CH,), jnp.float32),
        pltpu.SemaphoreType.DMA,
    ],
    compiler_params=pltpu.CompilerParams(use_tc_tiling_on_sc=False),
)
def _p3(inv_hbm, comb_hbm, norm_hbm, cbuf, nbuf, sem):
    def chunk(c):
        base = c * CH
        pltpu.sync_copy(comb_hbm.at[pl.ds(base, CH)], cbuf)
        pltpu.async_copy(inv_hbm.at[cbuf], nbuf, sem).wait()
        pltpu.sync_copy(nbuf, norm_hbm.at[pl.ds(base, CH)])

    _chunk_loop(chunk)


_TM = 1000


def _t_body(x_ref, w_ref, b_ref, out_ref):
    g = pl.program_id(0)
    y = jnp.dot(x_ref[...], w_ref[0], preferred_element_type=jnp.float32)
    is_root = (g == R).astype(jnp.float32)
    out_ref[0] = y + b_ref[...] * is_root


def _t(x, wall, b):
    return pl.pallas_call(
        _t_body,
        grid=(R + 1, N // _TM),
        in_specs=[
            pl.BlockSpec((_TM, H), lambda g, i: (i, 0)),
            pl.BlockSpec((1, H, H), lambda g, i: (g, 0, 0)),
            pl.BlockSpec((1, H), lambda g, i: (0, 0)),
        ],
        out_specs=pl.BlockSpec((1, _TM, H), lambda g, i: (g, i, 0)),
        out_shape=jax.ShapeDtypeStruct((R + 1, N, H), jnp.float32),
    )(x, wall, b.reshape(1, H))


@functools.partial(
    pl.kernel,
    out_type=jax.ShapeDtypeStruct((NC, N, H), jnp.float32),
    mesh=_MESH,
    scratch_types=[
        pltpu.VMEM((CH,), jnp.int32),
        pltpu.VMEM((CH,), jnp.int32),
        pltpu.VMEM((CH,), jnp.float32),
        pltpu.VMEM((CH, H), jnp.float32),
        pltpu.VMEM_SHARED((N, H), jnp.float32),
        pltpu.SemaphoreType.DMA,
    ],
    compiler_params=pltpu.CompilerParams(needs_layout_passes=False,
                                         use_tc_tiling_on_sc=False),
)
def _s(ytab_hbm, gidx_hbm, dst_hbm, norm_hbm, part_hbm,
       gbuf, dbuf, nbuf, rows, acc, sem):
    cid = lax.axis_index("c")
    sid = lax.axis_index("s")

    def zrow(r, carry):
        for k in range(HV):
            rows[r, pl.ds(k * LANES, LANES)] = jnp.zeros((LANES,), jnp.float32)
        return carry

    lax.fori_loop(0, CH, zrow, 0)
    arow = sid * 1000

    @pl.when(sid < 10)
    def _():
        def zcopy(i, carry):
            pltpu.sync_copy(rows.at[pl.ds(0, CH)],
                            acc.at[pl.ds(arow + i * CH, CH)])
            return carry

        lax.fori_loop(0, 7, zcopy, 0)
        pltpu.sync_copy(rows.at[pl.ds(0, 104)],
                        acc.at[pl.ds(arow + 7 * CH, 104)])

    plsc.subcore_barrier()

    def chunk(c):
        base = c * CH
        pltpu.sync_copy(gidx_hbm.at[pl.ds(base, CH)], gbuf)
        pltpu.sync_copy(dst_hbm.at[pl.ds(base, CH)], dbuf)
        pltpu.sync_copy(norm_hbm.at[pl.ds(base, CH)], nbuf)
        pltpu.async_copy(ytab_hbm.at[gbuf], rows, sem).wait()

        def scale(e, carry):
            nv = plsc.load_gather(nbuf, [jnp.broadcast_to(e, (LANES,))])
            for k in range(HV):
                sl = pl.ds(k * LANES, LANES)
                rows[e, sl] = rows[e, sl] * nv
            return carry

        lax.fori_loop(0, CH, scale, 0)
        pltpu.sync_copy(rows, acc.at[dbuf], add=True)

    _chunk_loop(chunk)
    plsc.subcore_barrier()

    @pl.when(sid < 10)
    def _():
        def dump(i, carry):
            pltpu.sync_copy(acc.at[pl.ds(arow + i * CH, CH)], rows)
            pltpu.sync_copy(rows, part_hbm.at[cid, pl.ds(arow + i * CH, CH)])
            return carry

        lax.fori_loop(0, 7, dump, 0)
        pltpu.sync_copy(acc.at[pl.ds(arow + 7 * CH, 104)],
                        rows.at[pl.ds(0, 104)])
        pltpu.sync_copy(rows.at[pl.ds(0, 104)],
                        part_hbm.at[cid, pl.ds(arow + 7 * CH, 104)])


def _c_body(relu, p_ref, z_ref, out_ref):
    y = p_ref[0] + p_ref[1] + z_ref[...]
    if relu:
        y = jnp.maximum(y, 0.0)
    out_ref[...] = y


def _c(parts, z, relu):
    return pl.pallas_call(
        functools.partial(_c_body, relu),
        grid=(N // _TM,),
        in_specs=[
            pl.BlockSpec((NC, _TM, H), lambda i: (0, i, 0)),
            pl.BlockSpec((_TM, H), lambda i: (i, 0)),
        ],
        out_specs=pl.BlockSpec((_TM, H), lambda i: (i, 0)),
        out_shape=jax.ShapeDtypeStruct((N, H), jnp.float32),
    )(parts, z)


def _block_diag_weights(w, root):
    wd = jnp.zeros((R, NB, BS, NB, BS), jnp.float32)
    idx = jnp.arange(NB)
    wd = wd.at[:, idx, :, idx, :].set(w.transpose(1, 0, 2, 3))
    wd = wd.reshape(R, H, H)
    return jnp.concatenate([wd, root[None]], axis=0)


def kernel(node_emb, w0, root0, b0, w1, root1, b1, edge_index, edge_type):
    src = edge_index[0]
    dst = edge_index[1]
    gidx, comb, cnt = _p1(src, dst, edge_type)
    inv = _p2(cnt)
    norm = _p3(inv, comb)

    wall0 = _block_diag_weights(w0, root0)
    wall1 = _block_diag_weights(w1, root1)

    x = node_emb
    for wall, b, relu in ((wall0, b0, True), (wall1, b1, True),
                          (wall1, b1, False)):
        ytab = _t(x, wall, b)
        parts = _s(ytab.reshape((R + 1) * N, H), gidx, dst, norm)
        x = _c(parts, ytab[R], relu)
    return x


# speedup vs baseline: 6.8609x; 1.2477x over previous
import functools

import jax
import jax.numpy as jnp
from jax import lax
from jax.experimental import pallas as pl
from jax.experimental.pallas import tpu as pltpu
from jax.experimental.pallas import tpu_sc as plsc

N = 10000
R = 8
H = 160
NB = 5
BS = H // NB
E = 320000
NR = N * R

NC = 2
NS = 16
NW = NC * NS
LANES = 16

CH = 128
NCH = E // CH
ROWS_PER_TILE = N // NS
HV = H // LANES

_MESH = plsc.VectorSubcoreMesh(core_axis_name="c", subcore_axis_name="s")


def _worker_id():
    return lax.axis_index("c") * NS + lax.axis_index("s")


def _chunk_loop(body):
    wid = _worker_id()
    n_my = (NCH - wid + NW - 1) // NW

    def outer(i, carry):
        body(wid + i * NW)
        return carry

    lax.fori_loop(0, n_my, outer, 0)


@functools.partial(
    pl.kernel,
    out_type=(
        jax.ShapeDtypeStruct((E,), jnp.int32),
        jax.ShapeDtypeStruct((E,), jnp.int32),
        jax.ShapeDtypeStruct((NC * NR,), jnp.float32),
    ),
    mesh=_MESH,
    scratch_types=[
        pltpu.VMEM((CH,), jnp.int32),
        pltpu.VMEM((CH,), jnp.int32),
        pltpu.VMEM((CH,), jnp.int32),
        pltpu.VMEM((CH,), jnp.int32),
        pltpu.VMEM((CH,), jnp.int32),
        pltpu.VMEM((CH,), jnp.float32),
        pltpu.VMEM((CH,), jnp.float32),
        pltpu.VMEM_SHARED((NR,), jnp.float32),
    ],
    compiler_params=pltpu.CompilerParams(use_tc_tiling_on_sc=False),
)
def _p1(src_hbm, dst_hbm, typ_hbm, gidx_hbm, comb_hbm, cnt_hbm,
        sbuf, dbuf, tbuf, gbuf, cbuf, ones, zbuf, cnt_sh):
    cid = lax.axis_index("c")
    sid = lax.axis_index("s")

    for v in range(CH // LANES):
        ones[pl.ds(v * LANES, LANES)] = jnp.ones((LANES,), jnp.float32)
        zbuf[pl.ds(v * LANES, LANES)] = jnp.zeros((LANES,), jnp.float32)
    zrow = sid * (NR // NS)

    def zero_strip(i, carry):
        pltpu.sync_copy(zbuf.at[pl.ds(0, CH)],
                        cnt_sh.at[pl.ds(zrow + i * CH, CH)])
        return carry

    lax.fori_loop(0, (NR // NS) // CH, zero_strip, 0)
    pltpu.sync_copy(zbuf.at[pl.ds(0, 8)],
                    cnt_sh.at[pl.ds(zrow + ((NR // NS) // CH) * CH, 8)])
    plsc.subcore_barrier()

    def chunk(c):
        base = c * CH
        pltpu.sync_copy(src_hbm.at[pl.ds(base, CH)], sbuf)
        pltpu.sync_copy(dst_hbm.at[pl.ds(base, CH)], dbuf)
        pltpu.sync_copy(typ_hbm.at[pl.ds(base, CH)], tbuf)

        def vec(v, carry):
            sl = pl.ds(v * LANES, LANES)
            s = sbuf[sl]
            d = dbuf[sl]
            t = tbuf[sl]
            gbuf[sl] = t * N + s
            cbuf[sl] = d * R + t
            return carry

        lax.fori_loop(0, CH // LANES, vec, 0)
        pltpu.sync_copy(gbuf, gidx_hbm.at[pl.ds(base, CH)])
        pltpu.sync_copy(cbuf, comb_hbm.at[pl.ds(base, CH)])
        pltpu.sync_copy(ones, cnt_sh.at[cbuf], add=True)

    _chunk_loop(chunk)
    plsc.subcore_barrier()

    def dump(i, carry):
        pltpu.sync_copy(cnt_sh.at[pl.ds(zrow + i * CH, CH)], zbuf)
        pltpu.sync_copy(zbuf, cnt_hbm.at[pl.ds(cid * NR + zrow + i * CH, CH)])
        return carry

    lax.fori_loop(0, (NR // NS) // CH, dump, 0)
    tail = ((NR // NS) // CH) * CH
    pltpu.sync_copy(cnt_sh.at[pl.ds(zrow + tail, 8)], zbuf.at[pl.ds(0, 8)])
    pltpu.sync_copy(zbuf.at[pl.ds(0, 8)],
                    cnt_hbm.at[pl.ds(cid * NR + zrow + tail, 8)])


def _p2_body(cnt_ref, inv_ref):
    c = cnt_ref[0] + cnt_ref[1]
    inv_ref[...] = 1.0 / jnp.maximum(c, 1.0)


def _p2(cnt):
    cnt3 = cnt.reshape(NC, NR // 128, 128)
    inv = pl.pallas_call(
        _p2_body,
        out_shape=jax.ShapeDtypeStruct((NR // 128, 128), jnp.float32),
    )(cnt3)
    return inv.reshape(NR)


@functools.partial(
    pl.kernel,
    out_type=jax.ShapeDtypeStruct((E,), jnp.float32),
    mesh=_MESH,
    scratch_types=[
        pltpu.VMEM((CH,), jnp.int32),
        pltpu.VMEM((CH,), jnp.float32),
        pltpu.SemaphoreType.DMA,
    ],
    compiler_params=pltpu.CompilerParams(use_tc_tiling_on_sc=False),
)
def _p3(inv_hbm, comb_hbm, norm_hbm, cbuf, nbuf, sem):
    def chunk(c):
        base = c * CH
        pltpu.sync_copy(comb_hbm.at[pl.ds(base, CH)], cbuf)
        pltpu.async_copy(inv_hbm.at[cbuf], nbuf, sem).wait()
        pltpu.sync_copy(nbuf, norm_hbm.at[pl.ds(base, CH)])

    _chunk_loop(chunk)


_TM = 1000


def _t_body(x_ref, w_ref, b_ref, out_ref):
    g = pl.program_id(0)
    y = jnp.dot(x_ref[...], w_ref[0], preferred_element_type=jnp.float32)
    is_root = (g == R).astype(jnp.float32)
    out_ref[0] = y + b_ref[...] * is_root


def _t(x, wall, b):
    return pl.pallas_call(
        _t_body,
        grid=(R + 1, N // _TM),
        in_specs=[
            pl.BlockSpec((_TM, H), lambda g, i: (i, 0)),
            pl.BlockSpec((1, H, H), lambda g, i: (g, 0, 0)),
            pl.BlockSpec((1, H), lambda g, i: (0, 0)),
        ],
        out_specs=pl.BlockSpec((1, _TM, H), lambda g, i: (g, i, 0)),
        out_shape=jax.ShapeDtypeStruct((R + 1, N, H), jnp.float32),
    )(x, wall, b.reshape(1, H))


SCH = 64
NCH_S = E // SCH
_NITER = (NCH_S + NW - 1) // NW


@functools.partial(
    pl.kernel,
    out_type=jax.ShapeDtypeStruct((NC, N, H), jnp.float32),
    mesh=_MESH,
    scratch_types=[
        pltpu.VMEM((SCH,), jnp.int32),
        pltpu.VMEM((SCH,), jnp.int32),
        pltpu.VMEM((SCH,), jnp.int32),
        pltpu.VMEM((SCH,), jnp.int32),
        pltpu.VMEM((SCH,), jnp.float32),
        pltpu.VMEM((SCH,), jnp.float32),
        pltpu.VMEM((SCH, H), jnp.float32),
        pltpu.VMEM((SCH, H), jnp.float32),
        pltpu.VMEM_SHARED((N, H), jnp.float32),
        pltpu.SemaphoreType.DMA,
        pltpu.SemaphoreType.DMA,
        pltpu.SemaphoreType.DMA,
        pltpu.SemaphoreType.DMA,
        pltpu.SemaphoreType.DMA,
        pltpu.SemaphoreType.DMA,
    ],
    compiler_params=pltpu.CompilerParams(needs_layout_passes=False,
                                         use_tc_tiling_on_sc=False),
)
def _s(ytab_hbm, gidx_hbm, dst_hbm, norm_hbm, part_hbm,
       g0, g1, d0, d1, n0, n1, r0, r1, acc, si0, si1, sg0, sg1, ss0, ss1):
    cid = lax.axis_index("c")
    sid = lax.axis_index("s")
    wid = cid * NS + sid
    G = (g0, g1)
    D = (d0, d1)
    NM = (n0, n1)
    RW = (r0, r1)
    SI = (si0, si1)
    SG = (sg0, sg1)
    SS = (ss0, ss1)

    def zrow(r, carry):
        for k in range(HV):
            r0[r, pl.ds(k * LANES, LANES)] = jnp.zeros((LANES,), jnp.float32)
        return carry

    lax.fori_loop(0, SCH, zrow, 0)
    arow = sid * 1000

    @pl.when(sid < 10)
    def _():
        def zcopy(i, carry):
            pltpu.sync_copy(r0.at[pl.ds(0, SCH)],
                            acc.at[pl.ds(arow + i * SCH, SCH)])
            return carry

        lax.fori_loop(0, 15, zcopy, 0)
        pltpu.sync_copy(r0.at[pl.ds(0, 40)],
                        acc.at[pl.ds(arow + 15 * SCH, 40)])

    plsc.subcore_barrier()

    def c_of(i):
        c = wid + i * NW
        return jnp.where(c >= NCH_S, wid, c)

    def is_pad(i):
        return (wid + i * NW) >= NCH_S

    def issue_idx(s, i):
        base = c_of(i) * SCH
        pltpu.async_copy(gidx_hbm.at[pl.ds(base, SCH)], G[s], SI[s])
        pltpu.async_copy(dst_hbm.at[pl.ds(base, SCH)], D[s], SI[s])
        pltpu.async_copy(norm_hbm.at[pl.ds(base, SCH)], NM[s], SI[s])

    def wait_idx(s):
        pltpu.make_async_copy(gidx_hbm.at[pl.ds(0, SCH)], G[s], SI[s]).wait()
        pltpu.make_async_copy(dst_hbm.at[pl.ds(0, SCH)], D[s], SI[s]).wait()
        pltpu.make_async_copy(norm_hbm.at[pl.ds(0, SCH)], NM[s], SI[s]).wait()

    def issue_gather(s):
        pltpu.async_copy(ytab_hbm.at[G[s]], RW[s], SG[s])

    def wait_gather(s):
        pltpu.make_async_copy(ytab_hbm.at[G[s]], RW[s], SG[s]).wait()

    def issue_scat(s):
        pltpu.async_copy(RW[s], acc.at[D[s]], SS[s], add=True)

    def wait_scat(s):
        pltpu.make_async_copy(RW[s], acc.at[D[s]], SS[s]).wait()

    def scale(s, i):
        @pl.when(is_pad(i))
        def _():
            for v in range(SCH // LANES):
                NM[s][pl.ds(v * LANES, LANES)] = jnp.zeros((LANES,),
                                                           jnp.float32)

        def body(e, carry):
            nv = plsc.load_gather(NM[s], [jnp.broadcast_to(e, (LANES,))])
            for k in range(HV):
                sl = pl.ds(k * LANES, LANES)
                RW[s][e, sl] = RW[s][e, sl] * nv
            return carry

        lax.fori_loop(0, SCH, body, 0)

    def half(i, s):
        t = 1 - s
        wait_scat(t)
        issue_idx(t, i + 1)
        wait_gather(s)
        wait_idx(t)
        issue_gather(t)
        scale(s, i)
        issue_scat(s)

    issue_idx(0, 0)
    wait_idx(0)
    issue_gather(0)
    issue_idx(1, 1)
    wait_gather(0)
    wait_idx(1)
    issue_gather(1)
    scale(0, 0)
    issue_scat(0)

    def pair(p, carry):
        half(1 + 2 * p, 1)
        half(2 + 2 * p, 0)
        return carry

    lax.fori_loop(0, (_NITER - 1) // 2, pair, 0)
    wait_gather(1)
    wait_scat(0)

    plsc.subcore_barrier()

    @pl.when(sid < 10)
    def _():
        def dump(i, carry):
            pltpu.sync_copy(acc.at[pl.ds(arow + i * SCH, SCH)], r0)
            pltpu.sync_copy(r0, part_hbm.at[cid, pl.ds(arow + i * SCH, SCH)])
            return carry

        lax.fori_loop(0, 15, dump, 0)
        pltpu.sync_copy(acc.at[pl.ds(arow + 15 * SCH, 40)],
                        r0.at[pl.ds(0, 40)])
        pltpu.sync_copy(r0.at[pl.ds(0, 40)],
                        part_hbm.at[cid, pl.ds(arow + 15 * SCH, 40)])


def _c_body(relu, p_ref, z_ref, out_ref):
    y = p_ref[0] + p_ref[1] + z_ref[...]
    if relu:
        y = jnp.maximum(y, 0.0)
    out_ref[...] = y


def _c(parts, z, relu):
    return pl.pallas_call(
        functools.partial(_c_body, relu),
        grid=(N // _TM,),
        in_specs=[
            pl.BlockSpec((NC, _TM, H), lambda i: (0, i, 0)),
            pl.BlockSpec((_TM, H), lambda i: (i, 0)),
        ],
        out_specs=pl.BlockSpec((_TM, H), lambda i: (i, 0)),
        out_shape=jax.ShapeDtypeStruct((N, H), jnp.float32),
    )(parts, z)


def _block_diag_weights(w, root):
    wd = jnp.zeros((R, NB, BS, NB, BS), jnp.float32)
    idx = jnp.arange(NB)
    wd = wd.at[:, idx, :, idx, :].set(w.transpose(1, 0, 2, 3))
    wd = wd.reshape(R, H, H)
    return jnp.concatenate([wd, root[None]], axis=0)


def kernel(node_emb, w0, root0, b0, w1, root1, b1, edge_index, edge_type):
    src = edge_index[0]
    dst = edge_index[1]
    gidx, comb, cnt = _p1(src, dst, edge_type)
    inv = _p2(cnt)
    norm = _p3(inv, comb)

    wall0 = _block_diag_weights(w0, root0)
    wall1 = _block_diag_weights(w1, root1)

    x = node_emb
    for wall, b, relu in ((wall0, b0, True), (wall1, b1, True),
                          (wall1, b1, False)):
        ytab = _t(x, wall, b)
        parts = _s(ytab.reshape((R + 1) * N, H), gidx, dst, norm)
        x = _c(parts, ytab[R], relu)
    return x
